# baseline (speedup 1.0000x reference)
import jax
import jax.numpy as jnp
from jax import lax
from jax.experimental import pallas as pl
from jax.experimental.pallas import tpu as pltpu
from jax.experimental.pallas import tpu_sc as plsc

_VOCAB = 32000
_D = 2048
_BATCH = 4
_SEQ = 4096
_N = _BATCH * _SEQ
_NW = 32
_PER_W = _N // _NW
_K = 16
_NCH = _PER_W // _K
_NBUF = 2
_G = _NBUF // 2


def _emb_body(ids_hbm, table_hbm, out_hbm, idx_v, *scratch):
    rows = scratch[:_NBUF]
    gsem = scratch[_NBUF:2 * _NBUF]
    ssem = scratch[2 * _NBUF:]

    cid = lax.axis_index("c")
    sid = lax.axis_index("s")
    wid = sid * 2 + cid
    base = wid * _PER_W

    def start_gather(ch, b):
        pltpu.make_async_copy(
            table_hbm.at[idx_v.at[ch]], rows[b], gsem[b]
        ).start()

    def wait_gather(ch, b):
        pltpu.make_async_copy(
            table_hbm.at[idx_v.at[ch]], rows[b], gsem[b]
        ).wait()

    def out_copy(ch, b):
        return pltpu.make_async_copy(
            rows[b], out_hbm.at[pl.ds(base + ch * _K, _K)], ssem[b]
        )

    pltpu.sync_copy(ids_hbm.at[wid], idx_v)

    for b in range(_G):
        start_gather(b, b)

    def body(t, carry):
        j = t * _NBUF
        for b in range(_NBUF):
            ch = j + b
            wait_gather(ch, b)
            out_copy(ch, b).start()

            pre = ch + _G
            bp = (b + _G) % _NBUF

            @pl.when(pre < _NCH)
            def _():
                @pl.when(pre >= _NBUF)
                def _():
                    out_copy(pre - _NBUF, bp).wait()

                start_gather(pre, bp)

        return carry

    lax.fori_loop(0, _NCH // _NBUF, body, 0)

    for b in range(_NBUF):
        ch = _NCH - _NBUF + b
        out_copy(ch, b).wait()


@jax.jit
def _emb_lookup(ids3, table):
    mesh = plsc.VectorSubcoreMesh(core_axis_name="c", subcore_axis_name="s")
    f = pl.kernel(
        _emb_body,
        out_type=jax.ShapeDtypeStruct((_N, _D), jnp.float32),
        mesh=mesh,
        scratch_types=(
            [pltpu.VMEM((_NCH, _K), jnp.int32)]
            + [pltpu.VMEM((_K, _D), jnp.float32) for _ in range(_NBUF)]
            + [pltpu.SemaphoreType.DMA for _ in range(2 * _NBUF)]
        ),
    )
    return f(ids3, table)


def kernel(input_ids, table):
    ids3 = jnp.reshape(input_ids.astype(jnp.int32), (_NW, _NCH, _K))
    out = _emb_lookup(ids3, table)
    return out.reshape(_BATCH, _SEQ, _D)


# speedup vs baseline: 1.0365x; 1.0365x over previous
import jax
import jax.numpy as jnp
from jax import lax
from jax.experimental import pallas as pl
from jax.experimental.pallas import tpu as pltpu
from jax.experimental.pallas import tpu_sc as plsc

_VOCAB = 32000
_D = 2048
_BATCH = 4
_SEQ = 4096
_N = _BATCH * _SEQ
_NW = 32
_PER_W = _N // _NW
_K = 8
_NCH = _PER_W // _K
_NBUF = 4
_G = 3


def _emb_body(ids_hbm, table_hbm, out_hbm, idx_v, *scratch):
    rows = scratch[:_NBUF]
    gsem = scratch[_NBUF:2 * _NBUF]
    ssem = scratch[2 * _NBUF:]

    cid = lax.axis_index("c")
    sid = lax.axis_index("s")
    wid = sid * 2 + cid
    base = wid * _PER_W

    def start_gather(ch, b):
        pltpu.make_async_copy(
            table_hbm.at[idx_v.at[ch]], rows[b], gsem[b]
        ).start()

    def wait_gather(ch, b):
        pltpu.make_async_copy(
            table_hbm.at[idx_v.at[ch]], rows[b], gsem[b]
        ).wait()

    def out_copy(ch, b):
        return pltpu.make_async_copy(
            rows[b], out_hbm.at[pl.ds(base + ch * _K, _K)], ssem[b]
        )

    pltpu.sync_copy(ids_hbm.at[wid], idx_v)

    for b in range(_G):
        start_gather(b, b)

    def body(t, carry):
        j = t * _NBUF
        for b in range(_NBUF):
            ch = j + b
            wait_gather(ch, b)
            out_copy(ch, b).start()

            pre = ch + _G
            bp = (b + _G) % _NBUF

            @pl.when(pre < _NCH)
            def _():
                @pl.when(pre >= _NBUF)
                def _():
                    out_copy(pre - _NBUF, bp).wait()

                start_gather(pre, bp)

        return carry

    lax.fori_loop(0, _NCH // _NBUF, body, 0)

    for b in range(_NBUF):
        ch = _NCH - _NBUF + b
        out_copy(ch, b).wait()


@jax.jit
def _emb_lookup(ids3, table):
    mesh = plsc.VectorSubcoreMesh(core_axis_name="c", subcore_axis_name="s")
    f = pl.kernel(
        _emb_body,
        out_type=jax.ShapeDtypeStruct((_N, _D), jnp.float32),
        mesh=mesh,
        scratch_types=(
            [pltpu.VMEM((_NCH, _K), jnp.int32)]
            + [pltpu.VMEM((_K, _D), jnp.float32) for _ in range(_NBUF)]
            + [pltpu.SemaphoreType.DMA for _ in range(2 * _NBUF)]
        ),
    )
    return f(ids3, table)


def kernel(input_ids, table):
    ids3 = jnp.reshape(input_ids.astype(jnp.int32), (_NW, _NCH, _K))
    out = _emb_lookup(ids3, table)
    return out.reshape(_BATCH, _SEQ, _D)


# speedup vs baseline: 1.0400x; 1.0034x over previous
import functools

import jax
import jax.numpy as jnp
from jax import lax
from jax.experimental import pallas as pl
from jax.experimental.pallas import tpu as pltpu
from jax.experimental.pallas import tpu_sc as plsc

_VOCAB = 32000
_D = 2048
_BATCH = 4
_SEQ = 4096
_N = _BATCH * _SEQ
_NW = 32
_PER_W = _N // _NW
_K = 8
_NCH = _PER_W // _K
_NBUF = 4


def _emb_body(ids_hbm, table_hbm, out_hbm, idx_v, rows0, rows1, rows2, rows3, g0, g1, g2, g3, s0, s1, s2, s3):
    cid = lax.axis_index("c")
    sid = lax.axis_index("s")
    wid = sid * 2 + cid
    base = wid * _PER_W

    rows = (rows0, rows1, rows2, rows3)
    gsem = (g0, g1, g2, g3)
    ssem = (s0, s1, s2, s3)

    pltpu.sync_copy(ids_hbm.at[wid], idx_v)

    for b in range(_NBUF):
        pltpu.make_async_copy(
            table_hbm.at[idx_v.at[b]], rows[b], gsem[b]
        ).start()

    def body(t, carry):
        j = t * _NBUF
        for b in range(_NBUF):
            ch = j + b
            pltpu.make_async_copy(
                table_hbm.at[idx_v.at[ch]], rows[b], gsem[b]
            ).wait()
            out_slice = out_hbm.at[pl.ds(base + ch * _K, _K)]
            pltpu.make_async_copy(rows[b], out_slice, ssem[b]).start()
            nxt = ch + _NBUF

            @pl.when(nxt < _NCH)
            def _():
                pltpu.make_async_copy(rows[b], out_slice, ssem[b]).wait()
                pltpu.make_async_copy(
                    table_hbm.at[idx_v.at[nxt]], rows[b], gsem[b]
                ).start()

        return carry

    lax.fori_loop(0, _NCH // _NBUF, body, 0)

    for b in range(_NBUF):
        ch = _NCH - _NBUF + b
        out_slice = out_hbm.at[pl.ds(base + ch * _K, _K)]
        pltpu.make_async_copy(rows[b], out_slice, ssem[b]).wait()


@jax.jit
def _emb_lookup(ids3, table):
    mesh = plsc.VectorSubcoreMesh(core_axis_name="c", subcore_axis_name="s")
    f = pl.kernel(
        _emb_body,
        out_type=jax.ShapeDtypeStruct((_N, _D), jnp.float32),
        mesh=mesh,
        scratch_types=[
            pltpu.VMEM((_NCH, _K), jnp.int32),
            pltpu.VMEM((_K, _D), jnp.float32),
            pltpu.VMEM((_K, _D), jnp.float32),
            pltpu.VMEM((_K, _D), jnp.float32),
            pltpu.VMEM((_K, _D), jnp.float32),
            pltpu.SemaphoreType.DMA,
            pltpu.SemaphoreType.DMA,
            pltpu.SemaphoreType.DMA,
            pltpu.SemaphoreType.DMA,
            pltpu.SemaphoreType.DMA,
            pltpu.SemaphoreType.DMA,
            pltpu.SemaphoreType.DMA,
            pltpu.SemaphoreType.DMA,
        ],
    )
    return f(ids3, table)


def kernel(input_ids, table):
    ids3 = jnp.reshape(input_ids.astype(jnp.int32), (_NW, _NCH, _K))
    out = _emb_lookup(ids3, table)
    return out.reshape(_BATCH, _SEQ, _D)


# speedup vs baseline: 1.0405x; 1.0005x over previous
import jax
import jax.numpy as jnp
from jax import lax
from jax.experimental import pallas as pl
from jax.experimental.pallas import tpu as pltpu
from jax.experimental.pallas import tpu_sc as plsc

_VOCAB = 32000
_D = 2048
_BATCH = 4
_SEQ = 4096
_N = _BATCH * _SEQ
_NW = 32
_PER_W = _N // _NW
_K = 8
_NCH = _PER_W // _K
_NBUF = 4


def _emb_body(ids_hbm, table_hbm, out_hbm, idx_v, rows0, rows1, rows2, rows3, g0, g1, g2, g3, s0, s1, s2, s3):
    cid = lax.axis_index("c")
    sid = lax.axis_index("s")
    wid = sid * 2 + cid
    base = wid * _PER_W

    rows = (rows0, rows1, rows2, rows3)
    gsem = (g0, g1, g2, g3)
    ssem = (s0, s1, s2, s3)

    pltpu.sync_copy(ids_hbm.at[wid], idx_v)

    for b in range(_NBUF):
        pltpu.make_async_copy(
            table_hbm.at[idx_v.at[b]], rows[b], gsem[b]
        ).start()

    def body(t, carry):
        j = t * _NBUF
        for b in range(_NBUF):
            ch = j + b
            pltpu.make_async_copy(
                table_hbm.at[idx_v.at[ch]], rows[b], gsem[b]
            ).wait()
            out_slice = out_hbm.at[pl.ds(base + ch * _K, _K)]
            pltpu.make_async_copy(rows[b], out_slice, ssem[b]).start()
            nxt = ch + _NBUF

            @pl.when(nxt < _NCH)
            def _():
                pltpu.make_async_copy(rows[b], out_slice, ssem[b]).wait()
                pltpu.make_async_copy(
                    table_hbm.at[idx_v.at[nxt]], rows[b], gsem[b]
                ).start()

        return carry

    lax.fori_loop(0, _NCH // _NBUF, body, 0)

    for b in range(_NBUF):
        ch = _NCH - _NBUF + b
        out_slice = out_hbm.at[pl.ds(base + ch * _K, _K)]
        pltpu.make_async_copy(rows[b], out_slice, ssem[b]).wait()


@jax.jit
def _emb_lookup(ids3, table):
    mesh = plsc.VectorSubcoreMesh(core_axis_name="c", subcore_axis_name="s")
    f = pl.kernel(
        _emb_body,
        out_type=jax.ShapeDtypeStruct((_N, _D), jnp.float32),
        mesh=mesh,
        scratch_types=[
            pltpu.VMEM((_NCH, _K), jnp.int32),
            pltpu.VMEM((_K, _D), jnp.float32),
            pltpu.VMEM((_K, _D), jnp.float32),
            pltpu.VMEM((_K, _D), jnp.float32),
            pltpu.VMEM((_K, _D), jnp.float32),
            pltpu.SemaphoreType.DMA,
            pltpu.SemaphoreType.DMA,
            pltpu.SemaphoreType.DMA,
            pltpu.SemaphoreType.DMA,
            pltpu.SemaphoreType.DMA,
            pltpu.SemaphoreType.DMA,
            pltpu.SemaphoreType.DMA,
            pltpu.SemaphoreType.DMA,
        ],
    )
    return f(ids3, table)


def kernel(input_ids, table):
    ids3 = jnp.reshape(input_ids.astype(jnp.int32), (_NW, _NCH, _K))
    out = _emb_lookup(ids3, table)
    return out.reshape(_BATCH, _SEQ, _D)
